# baseline (speedup 1.0000x reference)
import functools

import jax
import jax.numpy as jnp
from jax.experimental import pallas as pl
from jax.experimental.pallas import tpu as pltpu

E = 64
DIM = 2048
BETA = 0.5
BT = 2048
KC = 1024
NK = DIM // KC


def _gate_body(z_ref, wa_ref, wb_ref, s_ref, idx_ref, cosa_ref, cosb_ref,
               acca_ref, accb_ref):
    k = pl.program_id(1)
    zb = z_ref[...]
    pa = jnp.dot(zb, wa_ref[...], preferred_element_type=jnp.float32)
    pb = jnp.dot(zb, wb_ref[...], preferred_element_type=jnp.float32)

    @pl.when(k == 0)
    def _init():
        acca_ref[...] = pa
        accb_ref[...] = pb

    @pl.when(k > 0)
    def _accum():
        acca_ref[...] += pa
        accb_ref[...] += pb

    @pl.when(k == NK - 1)
    def _epilogue():
        ca = jnp.clip(acca_ref[...], -1.0, 1.0)
        cb = jnp.clip(accb_ref[...], -1.0, 1.0)
        s = ca - BETA * cb
        cosa_ref[...] = ca
        cosb_ref[...] = cb
        s_ref[...] = s

        iota = jax.lax.broadcasted_iota(jnp.int32, s.shape, 1)
        m1 = jnp.max(s, axis=1, keepdims=True)
        i1 = jnp.min(jnp.where(s == m1, iota, E), axis=1, keepdims=True)
        s2 = jnp.where(iota == i1, -jnp.inf, s)
        m2 = jnp.max(s2, axis=1, keepdims=True)
        i2 = jnp.min(jnp.where(s2 == m2, iota, E), axis=1, keepdims=True)
        idx_ref[...] = jnp.concatenate([i1, i2], axis=1)


@jax.jit
def kernel(z, A, B):
    ntok = z.shape[0]
    wa = A.T
    wb = B.T
    grid = (ntok // BT, NK)
    s, idx, ca, cb = pl.pallas_call(
        _gate_body,
        grid=grid,
        in_specs=[
            pl.BlockSpec((BT, KC), lambda i, k: (i, k)),
            pl.BlockSpec((KC, E), lambda i, k: (k, 0)),
            pl.BlockSpec((KC, E), lambda i, k: (k, 0)),
        ],
        out_specs=[
            pl.BlockSpec((BT, E), lambda i, k: (i, 0)),
            pl.BlockSpec((BT, 2), lambda i, k: (i, 0)),
            pl.BlockSpec((BT, E), lambda i, k: (i, 0)),
            pl.BlockSpec((BT, E), lambda i, k: (i, 0)),
        ],
        out_shape=[
            jax.ShapeDtypeStruct((ntok, E), jnp.float32),
            jax.ShapeDtypeStruct((ntok, 2), jnp.int32),
            jax.ShapeDtypeStruct((ntok, E), jnp.float32),
            jax.ShapeDtypeStruct((ntok, E), jnp.float32),
        ],
        scratch_shapes=[
            pltpu.VMEM((BT, E), jnp.float32),
            pltpu.VMEM((BT, E), jnp.float32),
        ],
        compiler_params=pltpu.CompilerParams(
            dimension_semantics=("arbitrary", "arbitrary"),
        ),
    )(z, wa, wb)
    return (s, idx, ca, cb)


# speedup vs baseline: 1.2423x; 1.2423x over previous
import functools

import jax
import jax.numpy as jnp
from jax.experimental import pallas as pl
from jax.experimental.pallas import tpu as pltpu

E = 64
DIM = 2048
KH = DIM // 2
BETA = 0.5
BT = 2048


def _gate_body(z1_ref, z2_ref, wa1_ref, wa2_ref, wb1_ref, wb2_ref,
               s_ref, idx_ref, cosa_ref, cosb_ref):
    z1 = z1_ref[...]
    z2 = z2_ref[...]
    ca = jnp.dot(z1, wa1_ref[...], preferred_element_type=jnp.float32)
    ca = ca + jnp.dot(z2, wa2_ref[...], preferred_element_type=jnp.float32)
    cb = jnp.dot(z1, wb1_ref[...], preferred_element_type=jnp.float32)
    cb = cb + jnp.dot(z2, wb2_ref[...], preferred_element_type=jnp.float32)
    ca = jnp.clip(ca, -1.0, 1.0)
    cb = jnp.clip(cb, -1.0, 1.0)
    s = ca - BETA * cb
    cosa_ref[...] = ca
    cosb_ref[...] = cb
    s_ref[...] = s

    iota = jax.lax.broadcasted_iota(jnp.int32, s.shape, 1)
    m1 = jnp.max(s, axis=1, keepdims=True)
    i1 = jnp.min(jnp.where(s == m1, iota, E), axis=1, keepdims=True)
    s2 = jnp.where(iota == i1, -jnp.inf, s)
    m2 = jnp.max(s2, axis=1, keepdims=True)
    i2 = jnp.min(jnp.where(s2 == m2, iota, E), axis=1, keepdims=True)
    idx_ref[...] = jnp.concatenate([i1, i2], axis=1)


@jax.jit
def kernel(z, A, B):
    ntok = z.shape[0]
    wa = A.T
    wb = B.T
    grid = (ntok // BT,)
    s, idx, ca, cb = pl.pallas_call(
        _gate_body,
        grid=grid,
        in_specs=[
            pl.BlockSpec((BT, KH), lambda i: (i, 0)),
            pl.BlockSpec((BT, KH), lambda i: (i, 1)),
            pl.BlockSpec((KH, E), lambda i: (0, 0)),
            pl.BlockSpec((KH, E), lambda i: (1, 0)),
            pl.BlockSpec((KH, E), lambda i: (0, 0)),
            pl.BlockSpec((KH, E), lambda i: (1, 0)),
        ],
        out_specs=[
            pl.BlockSpec((BT, E), lambda i: (i, 0)),
            pl.BlockSpec((BT, 2), lambda i: (i, 0)),
            pl.BlockSpec((BT, E), lambda i: (i, 0)),
            pl.BlockSpec((BT, E), lambda i: (i, 0)),
        ],
        out_shape=[
            jax.ShapeDtypeStruct((ntok, E), jnp.float32),
            jax.ShapeDtypeStruct((ntok, 2), jnp.int32),
            jax.ShapeDtypeStruct((ntok, E), jnp.float32),
            jax.ShapeDtypeStruct((ntok, E), jnp.float32),
        ],
        compiler_params=pltpu.CompilerParams(
            dimension_semantics=("arbitrary",),
        ),
    )(z, z, wa, wa, wb, wb)
    return (s, idx, ca, cb)
